# Initial kernel scaffold; baseline (speedup 1.0000x reference)
import jax
import jax.numpy as jnp
from jax.experimental import pallas as pl


def kernel(news_batch, news_id, news_repr_table, news_embedding_table):
    raise NotImplementedError("write your pallas kernel here")


# speedup vs baseline: 1.1538x; 1.1538x over previous
import functools

import jax
import jax.numpy as jnp
from jax import lax
from jax.experimental import pallas as pl
from jax.experimental.pallas import tpu as pltpu
from jax.experimental.pallas import tpu_sc as plsc

LEVEL = 16
HIDDEN = 32
EMB_D = LEVEL * HIDDEN

NUM_CORES = 2
NUM_SUBCORES = 16
NW = NUM_CORES * NUM_SUBCORES


def _make_gather(batch, nnews):
    total = batch * nnews
    assert total % NW == 0
    bpw = total // NW
    chunk = 80
    assert bpw % chunk == 0
    nch = bpw // chunk

    mesh = plsc.VectorSubcoreMesh(core_axis_name="c", subcore_axis_name="s")

    @functools.partial(
        pl.kernel,
        mesh=mesh,
        compiler_params=pltpu.CompilerParams(use_tc_tiling_on_sc=False),
        out_type=(
            jax.ShapeDtypeStruct((total, EMB_D), jnp.float32),
            jax.ShapeDtypeStruct((total, HIDDEN), jnp.float32),
        ),
        scratch_types=[
            pltpu.VMEM((nch, chunk), jnp.int32),
            pltpu.VMEM((chunk, EMB_D), jnp.float32),
            pltpu.VMEM((chunk, HIDDEN), jnp.float32),
            pltpu.SemaphoreType.DMA,
        ],
    )
    def gather_kernel(idx_hbm, emb_hbm, repr_hbm, out_emb, out_repr,
                      idx_v, ebuf, rbuf, sem):
        wid = lax.axis_index("s") * NUM_CORES + lax.axis_index("c")
        base = wid * bpw
        pltpu.sync_copy(idx_hbm.at[wid], idx_v)

        def body(j, carry):
            row0 = pl.multiple_of(base + j * chunk, chunk)
            pltpu.async_copy(emb_hbm.at[idx_v.at[j]], ebuf, sem).wait()
            pltpu.sync_copy(ebuf, out_emb.at[pl.ds(row0, chunk)])
            pltpu.async_copy(repr_hbm.at[idx_v.at[j]], rbuf, sem).wait()
            pltpu.sync_copy(rbuf, out_repr.at[pl.ds(row0, chunk)])
            return carry

        lax.fori_loop(0, nch, body, 0)

    return gather_kernel, nch, chunk


def kernel(news_batch, news_id, news_repr_table, news_embedding_table):
    batch, nnews = news_id.shape
    gather_kernel, nch, chunk = _make_gather(batch, nnews)
    idx = news_id.astype(jnp.int32).reshape(NW, nch, chunk)
    out_emb, out_repr = gather_kernel(
        idx, news_embedding_table, news_repr_table)
    news_embedding = out_emb.reshape(batch, nnews, LEVEL, HIDDEN)
    news_repr = out_repr.reshape(batch, nnews, HIDDEN)
    return (news_embedding, news_repr)


# speedup vs baseline: 1.1965x; 1.0370x over previous
import functools

import jax
import jax.numpy as jnp
from jax import lax
from jax.experimental import pallas as pl
from jax.experimental.pallas import tpu as pltpu
from jax.experimental.pallas import tpu_sc as plsc

LEVEL = 16
HIDDEN = 32
EMB_D = LEVEL * HIDDEN

NUM_CORES = 2
NUM_SUBCORES = 16
NW = NUM_CORES * NUM_SUBCORES

CHUNK = 80
NBUF = 2


def _make_gather(batch, nnews):
    total = batch * nnews
    assert total % NW == 0
    bpw = total // NW
    assert bpw % (CHUNK * NBUF) == 0
    nch = bpw // CHUNK
    ngroups = nch // NBUF

    mesh = plsc.VectorSubcoreMesh(core_axis_name="c", subcore_axis_name="s")

    @functools.partial(
        pl.kernel,
        mesh=mesh,
        compiler_params=pltpu.CompilerParams(use_tc_tiling_on_sc=False),
        out_type=(
            jax.ShapeDtypeStruct((total, EMB_D), jnp.float32),
            jax.ShapeDtypeStruct((total, HIDDEN), jnp.float32),
        ),
        scratch_types=(
            [pltpu.VMEM((nch, CHUNK), jnp.int32)]
            + [pltpu.VMEM((CHUNK, EMB_D), jnp.float32) for _ in range(NBUF)]
            + [pltpu.VMEM((CHUNK, HIDDEN), jnp.float32) for _ in range(NBUF)]
            + [pltpu.SemaphoreType.DMA for _ in range(4 * NBUF)]
        ),
    )
    def gather_kernel(idx_hbm, emb_hbm, repr_hbm, out_emb, out_repr,
                      idx_v, *scratch):
        ebufs = scratch[:NBUF]
        rbufs = scratch[NBUF:2 * NBUF]
        sems = scratch[2 * NBUF:]
        gse = sems[:NBUF]
        gsr = sems[NBUF:2 * NBUF]
        wse = sems[2 * NBUF:3 * NBUF]
        wsr = sems[3 * NBUF:]

        wid = lax.axis_index("s") * NUM_CORES + lax.axis_index("c")
        base = wid * bpw
        pltpu.sync_copy(idx_hbm.at[wid], idx_v)

        def emb_gather(j, b):
            return pltpu.make_async_copy(
                emb_hbm.at[idx_v.at[j]], ebufs[b], gse[b])

        def repr_gather(j, b):
            return pltpu.make_async_copy(
                repr_hbm.at[idx_v.at[j]], rbufs[b], gsr[b])

        for b in range(NBUF):
            emb_gather(b, b).start()
            repr_gather(b, b).start()

        def group(g, carry):
            for b in range(NBUF):
                j = g * NBUF + b
                emb_gather(j, b).wait()
                repr_gather(j, b).wait()
                row0 = pl.multiple_of(base + j * CHUNK, CHUNK)
                we = pltpu.make_async_copy(
                    ebufs[b], out_emb.at[pl.ds(row0, CHUNK)], wse[b])
                wr = pltpu.make_async_copy(
                    rbufs[b], out_repr.at[pl.ds(row0, CHUNK)], wsr[b])
                we.start()
                wr.start()
                we.wait()
                wr.wait()

                @pl.when(g < ngroups - 1)
                def _():
                    emb_gather(j + NBUF, b).start()
                    repr_gather(j + NBUF, b).start()
            return carry

        lax.fori_loop(0, ngroups, group, 0)

    return gather_kernel, nch


def kernel(news_batch, news_id, news_repr_table, news_embedding_table):
    batch, nnews = news_id.shape
    gather_kernel, nch = _make_gather(batch, nnews)
    idx = news_id.astype(jnp.int32).reshape(NW, nch, CHUNK)
    out_emb, out_repr = gather_kernel(
        idx, news_embedding_table, news_repr_table)
    news_embedding = out_emb.reshape(batch, nnews, LEVEL, HIDDEN)
    news_repr = out_repr.reshape(batch, nnews, HIDDEN)
    return (news_embedding, news_repr)


# speedup vs baseline: 1.4572x; 1.2179x over previous
import functools

import jax
import jax.numpy as jnp
from jax import lax
from jax.experimental import pallas as pl
from jax.experimental.pallas import tpu as pltpu
from jax.experimental.pallas import tpu_sc as plsc

LEVEL = 16
HIDDEN = 32
EMB_D = LEVEL * HIDDEN

NUM_CORES = 2
NUM_SUBCORES = 16
NW = NUM_CORES * NUM_SUBCORES

CHUNK = 80
NBUF = 2


def _make_gather(total, width, tc_tiling):
    assert total % NW == 0
    bpw = total // NW
    assert bpw % (CHUNK * NBUF) == 0
    nch = bpw // CHUNK
    ngroups = nch // NBUF

    mesh = plsc.VectorSubcoreMesh(core_axis_name="c", subcore_axis_name="s")

    @functools.partial(
        pl.kernel,
        mesh=mesh,
        compiler_params=pltpu.CompilerParams(use_tc_tiling_on_sc=tc_tiling),
        out_type=jax.ShapeDtypeStruct((total, width), jnp.float32),
        scratch_types=(
            [pltpu.VMEM((bpw,), jnp.int32)]
            + [pltpu.VMEM((CHUNK, width), jnp.float32) for _ in range(NBUF)]
            + [pltpu.SemaphoreType.DMA for _ in range(2 * NBUF)]
        ),
    )
    def gather_kernel(idx_hbm, table_hbm, out_hbm, idx_v, *scratch):
        bufs = scratch[:NBUF]
        gsem = scratch[NBUF:2 * NBUF]
        wsem = scratch[2 * NBUF:]

        wid = lax.axis_index("s") * NUM_CORES + lax.axis_index("c")
        base = wid * bpw
        pltpu.sync_copy(idx_hbm.at[pl.ds(pl.multiple_of(base, bpw), bpw)],
                        idx_v)

        def gather(j, b):
            off = pl.multiple_of(j * CHUNK, CHUNK)
            return pltpu.make_async_copy(
                table_hbm.at[idx_v.at[pl.ds(off, CHUNK)]], bufs[b], gsem[b])

        for b in range(NBUF):
            gather(b, b).start()

        def group(g, carry):
            for b in range(NBUF):
                j = g * NBUF + b
                gather(j, b).wait()
                row0 = pl.multiple_of(base + j * CHUNK, CHUNK)
                w = pltpu.make_async_copy(
                    bufs[b], out_hbm.at[pl.ds(row0, CHUNK)], wsem[b])
                w.start()
                w.wait()

                @pl.when(g < ngroups - 1)
                def _():
                    gather(j + NBUF, b).start()
            return carry

        lax.fori_loop(0, ngroups, group, 0)

    return gather_kernel


def kernel(news_batch, news_id, news_repr_table, news_embedding_table):
    batch, nnews = news_id.shape
    total = batch * nnews
    idx = news_id.astype(jnp.int32).reshape(total)
    emb_gather = _make_gather(total, EMB_D, tc_tiling=True)
    repr_gather = _make_gather(total, HIDDEN, tc_tiling=False)
    out_emb = emb_gather(idx, news_embedding_table)
    out_repr = repr_gather(idx, news_repr_table)
    news_embedding = out_emb.reshape(batch, nnews, LEVEL, HIDDEN)
    news_repr = out_repr.reshape(batch, nnews, HIDDEN)
    return (news_embedding, news_repr)
